# Initial kernel scaffold; baseline (speedup 1.0000x reference)
import jax
import jax.numpy as jnp
from jax.experimental import pallas as pl


def kernel(v, vectors):
    raise NotImplementedError("write your pallas kernel here")


# speedup vs baseline: 5.1966x; 5.1966x over previous
import functools

import jax
import jax.numpy as jnp
from jax.experimental import pallas as pl
from jax.experimental.pallas import tpu as pltpu

N = 1024
D = 64
B_TOTAL = 8 * 576
GRID = 8
BLK = B_TOTAL // GRID
EMPTY_VAL = 1.0 / D


def _vq_tc_kernel(v_ref, w_ref, out_ref, idx_ref, loss_ref):
    i = pl.program_id(0)
    vb = v_ref[...]
    w = w_ref[...]

    hp = jax.lax.Precision.HIGHEST
    wn_row = jax.lax.dot_general(
        jnp.ones((1, D), jnp.float32), w * w,
        (((1,), (1,)), ((), ())), precision=hp,
        preferred_element_type=jnp.float32)
    s = jax.lax.dot_general(
        vb, w, (((1,), (1,)), ((), ())), precision=hp,
        preferred_element_type=jnp.float32)
    score = wn_row - 2.0 * s

    col = jax.lax.broadcasted_iota(jnp.int32, (BLK, N), 1)
    score = jnp.where(col == 0, jnp.float32(3e38), score)
    m = jnp.min(score, axis=1, keepdims=True)
    cand = jnp.where(score == m, col, jnp.int32(2**30))
    idx = jnp.min(cand, axis=1, keepdims=True)
    nonempty = jnp.any(vb != EMPTY_VAL, axis=1, keepdims=True)
    idx = jnp.where(nonempty, idx, jnp.int32(0))

    onehot = (col == idx).astype(jnp.float32)
    outb = jax.lax.dot_general(
        onehot, w, (((1,), (0,)), ((), ())), precision=hp,
        preferred_element_type=jnp.float32)

    out_ref[...] = outb
    idx_ref[...] = idx

    part = jnp.sum((outb - vb) ** 2)

    @pl.when(i == 0)
    def _():
        loss_ref[...] = jnp.zeros((1, 1), jnp.float32)

    loss_ref[...] = loss_ref[...] + part

    @pl.when(i == GRID - 1)
    def _():
        loss_ref[...] = loss_ref[...] / jnp.float32(B_TOTAL * D)


@functools.partial(jax.jit, static_argnames=("interpret",))
def _vq_tc(vf, vectors, interpret=False):
    out, idx, loss = pl.pallas_call(
        _vq_tc_kernel,
        grid=(GRID,),
        in_specs=[
            pl.BlockSpec((BLK, D), lambda i: (i, 0)),
            pl.BlockSpec((N, D), lambda i: (0, 0)),
        ],
        out_specs=[
            pl.BlockSpec((BLK, D), lambda i: (i, 0)),
            pl.BlockSpec((BLK, 1), lambda i: (i, 0)),
            pl.BlockSpec((1, 1), lambda i: (0, 0)),
        ],
        out_shape=[
            jax.ShapeDtypeStruct((B_TOTAL, D), jnp.float32),
            jax.ShapeDtypeStruct((B_TOTAL, 1), jnp.int32),
            jax.ShapeDtypeStruct((1, 1), jnp.float32),
        ],
        compiler_params=pltpu.CompilerParams(
            dimension_semantics=("arbitrary",)),
        interpret=interpret,
    )(vf, vectors)
    return out, idx, loss


def kernel(v, vectors, interpret=False):
    lead = v.shape[:-1]
    vf = v.reshape(-1, D)
    out, idx, loss = _vq_tc(vf, vectors, interpret=interpret)
    used = jnp.array([0], dtype=jnp.int32)
    return (out.reshape(*lead, D), idx.reshape(lead), loss[0, 0], used)


# speedup vs baseline: 6.2057x; 1.1942x over previous
import functools

import jax
import jax.numpy as jnp
from jax import lax
from jax.experimental import pallas as pl
from jax.experimental.pallas import tpu as pltpu
from jax.experimental.pallas import tpu_sc as plsc

N = 1024
D = 64
B_TOTAL = 8 * 576
GRID = 8
BLK = B_TOTAL // GRID
EMPTY_VAL = 1.0 / D

NC = 2
NS = 16
NW = NC * NS
BPW = B_TOTAL // NW
CH = BPW // 2
LANES = 16


def _vq_tc_kernel(v_ref, w_ref, idx_ref):
    vb = v_ref[...]
    w = w_ref[...]

    hp = jax.lax.Precision.HIGHEST
    wn_row = jax.lax.dot_general(
        jnp.ones((1, D), jnp.float32), w * w,
        (((1,), (1,)), ((), ())), precision=hp,
        preferred_element_type=jnp.float32)
    s = jax.lax.dot_general(
        vb, w, (((1,), (1,)), ((), ())), precision=hp,
        preferred_element_type=jnp.float32)
    score = wn_row - 2.0 * s

    col = jax.lax.broadcasted_iota(jnp.int32, (BLK, N), 1)
    score = jnp.where(col == 0, jnp.float32(3e38), score)
    m = jnp.min(score, axis=1, keepdims=True)
    cand = jnp.where(score == m, col, jnp.int32(2**30))
    idx = jnp.min(cand, axis=1, keepdims=True)
    nonempty = jnp.any(vb != EMPTY_VAL, axis=1, keepdims=True)
    idx_ref[...] = jnp.where(nonempty, idx, jnp.int32(0))


@functools.partial(
    pl.kernel,
    out_type=[
        jax.ShapeDtypeStruct((B_TOTAL, D), jnp.float32),
        jax.ShapeDtypeStruct((NW, LANES), jnp.float32),
    ],
    mesh=plsc.VectorSubcoreMesh(core_axis_name="c", subcore_axis_name="s"),
    compiler_params=pltpu.CompilerParams(use_tc_tiling_on_sc=False),
    scratch_types=[
        pltpu.VMEM((CH,), jnp.int32),
        pltpu.VMEM((CH,), jnp.int32),
        pltpu.VMEM((BPW, D), jnp.float32),
        pltpu.VMEM((BPW * D,), jnp.float32),
        pltpu.VMEM((LANES,), jnp.float32),
        pltpu.SemaphoreType.DMA,
        pltpu.SemaphoreType.DMA,
    ],
)
def _vq_sc_gather(table_hbm, idx_hbm, vflat_hbm, out_hbm, losspart_hbm,
                  idx_v0, idx_v1, rows_v, v_v, acc_v, sem0, sem1):
    wid = lax.axis_index("s") * NC + lax.axis_index("c")
    base = wid * BPW
    pltpu.sync_copy(idx_hbm.at[pl.ds(base, CH)], idx_v0)
    pltpu.sync_copy(idx_hbm.at[pl.ds(base + CH, CH)], idx_v1)
    cp0 = pltpu.async_copy(table_hbm.at[idx_v0], rows_v.at[pl.ds(0, CH)], sem0)
    cp1 = pltpu.async_copy(table_hbm.at[idx_v1], rows_v.at[pl.ds(CH, CH)],
                           sem1)
    pltpu.sync_copy(vflat_hbm.at[pl.ds(base * D, BPW * D)], v_v)
    cp0.wait()
    cp1.wait()
    pltpu.sync_copy(rows_v, out_hbm.at[pl.ds(base, BPW)])

    def body(i, acc):
        for j in range(D // LANES):
            r = rows_v[i, pl.ds(j * LANES, LANES)]
            vv = v_v[pl.ds(i * D + j * LANES, LANES)]
            d = r - vv
            acc = acc + d * d
        return acc

    acc = lax.fori_loop(0, BPW, body, jnp.zeros((LANES,), jnp.float32))
    acc_v[...] = acc
    pltpu.sync_copy(acc_v, losspart_hbm.at[wid])


@jax.jit
def _vq(vf, vectors):
    idx = pl.pallas_call(
        _vq_tc_kernel,
        grid=(GRID,),
        in_specs=[
            pl.BlockSpec((BLK, D), lambda i: (i, 0)),
            pl.BlockSpec((N, D), lambda i: (0, 0)),
        ],
        out_specs=pl.BlockSpec((BLK, 1), lambda i: (i, 0)),
        out_shape=jax.ShapeDtypeStruct((B_TOTAL, 1), jnp.int32),
        compiler_params=pltpu.CompilerParams(
            dimension_semantics=("arbitrary",)),
    )(vf, vectors)
    idx_flat = idx.reshape(B_TOTAL)
    out, losspart = _vq_sc_gather(vectors, idx_flat, vf.reshape(-1))
    loss = jnp.sum(losspart) / jnp.float32(B_TOTAL * D)
    return out, idx_flat, loss


def kernel(v, vectors):
    lead = v.shape[:-1]
    vf = v.reshape(-1, D)
    out, idx, loss = _vq(vf, vectors)
    used = jnp.array([0], dtype=jnp.int32)
    return (out.reshape(*lead, D), idx.reshape(lead), loss, used)


# speedup vs baseline: 6.2710x; 1.0105x over previous
import functools

import jax
import jax.numpy as jnp
from jax import lax
from jax.experimental import pallas as pl
from jax.experimental.pallas import tpu as pltpu
from jax.experimental.pallas import tpu_sc as plsc

N = 1024
D = 64
B_TOTAL = 8 * 576
GRID = 8
BLK = B_TOTAL // GRID
EMPTY_VAL = 1.0 / D

NC = 2
NS = 16
NW = NC * NS
BPW = B_TOTAL // NW
CH = BPW // 2
LANES = 16


def _vq_tc_kernel(v_ref, w_ref, idx_ref):
    vb = v_ref[...]
    w = w_ref[...]

    hp = jax.lax.Precision.HIGHEST
    wn_row = jax.lax.dot_general(
        jnp.ones((1, D), jnp.float32), w * w,
        (((1,), (1,)), ((), ())), precision=hp,
        preferred_element_type=jnp.float32)
    col0 = jax.lax.broadcasted_iota(jnp.int32, (1, N), 1)
    wn_row = jnp.where(col0 == 0, jnp.float32(3e38), wn_row)
    s = jax.lax.dot_general(
        vb * -2.0, w, (((1,), (1,)), ((), ())), precision=hp,
        preferred_element_type=jnp.float32)
    score = s + wn_row

    idx = jnp.argmin(score, axis=1).astype(jnp.int32)
    nonempty = jnp.any(vb != EMPTY_VAL, axis=1)
    idx_ref[...] = jnp.where(nonempty, idx, jnp.int32(0))[:, None]


@functools.partial(
    pl.kernel,
    out_type=[
        jax.ShapeDtypeStruct((B_TOTAL, D), jnp.float32),
        jax.ShapeDtypeStruct((NW, LANES), jnp.float32),
    ],
    mesh=plsc.VectorSubcoreMesh(core_axis_name="c", subcore_axis_name="s"),
    compiler_params=pltpu.CompilerParams(use_tc_tiling_on_sc=False),
    scratch_types=[
        pltpu.VMEM((CH,), jnp.int32),
        pltpu.VMEM((CH,), jnp.int32),
        pltpu.VMEM((BPW, D), jnp.float32),
        pltpu.VMEM((BPW * D,), jnp.float32),
        pltpu.VMEM((LANES,), jnp.float32),
        pltpu.SemaphoreType.DMA,
        pltpu.SemaphoreType.DMA,
    ],
)
def _vq_sc_gather(table_hbm, idx_hbm, vflat_hbm, out_hbm, losspart_hbm,
                  idx_v0, idx_v1, rows_v, v_v, acc_v, sem0, sem1):
    wid = lax.axis_index("s") * NC + lax.axis_index("c")
    base = wid * BPW
    pltpu.sync_copy(idx_hbm.at[pl.ds(base, CH)], idx_v0)
    pltpu.sync_copy(idx_hbm.at[pl.ds(base + CH, CH)], idx_v1)
    cp0 = pltpu.async_copy(table_hbm.at[idx_v0], rows_v.at[pl.ds(0, CH)], sem0)
    cp1 = pltpu.async_copy(table_hbm.at[idx_v1], rows_v.at[pl.ds(CH, CH)],
                           sem1)
    pltpu.sync_copy(vflat_hbm.at[pl.ds(base * D, BPW * D)], v_v)
    cp0.wait()
    cp1.wait()
    pltpu.sync_copy(rows_v, out_hbm.at[pl.ds(base, BPW)])

    def body(i, acc):
        for j in range(D // LANES):
            r = rows_v[i, pl.ds(j * LANES, LANES)]
            vv = v_v[pl.ds(i * D + j * LANES, LANES)]
            d = r - vv
            acc = acc + d * d
        return acc

    acc = lax.fori_loop(0, BPW, body, jnp.zeros((LANES,), jnp.float32))
    acc_v[...] = acc
    pltpu.sync_copy(acc_v, losspart_hbm.at[wid])


@jax.jit
def _vq(vf, vectors):
    idx = pl.pallas_call(
        _vq_tc_kernel,
        grid=(GRID,),
        in_specs=[
            pl.BlockSpec((BLK, D), lambda i: (i, 0)),
            pl.BlockSpec((N, D), lambda i: (0, 0)),
        ],
        out_specs=pl.BlockSpec((BLK, 1), lambda i: (i, 0)),
        out_shape=jax.ShapeDtypeStruct((B_TOTAL, 1), jnp.int32),
        compiler_params=pltpu.CompilerParams(
            dimension_semantics=("arbitrary",)),
    )(vf, vectors)
    idx_flat = idx.reshape(B_TOTAL)
    out, losspart = _vq_sc_gather(vectors, idx_flat, vf.reshape(-1))
    loss = jnp.sum(losspart) / jnp.float32(B_TOTAL * D)
    return out, idx_flat, loss


def kernel(v, vectors):
    lead = v.shape[:-1]
    vf = v.reshape(-1, D)
    out, idx, loss = _vq(vf, vectors)
    used = jnp.array([0], dtype=jnp.int32)
    return (out.reshape(*lead, D), idx.reshape(lead), loss, used)
